# baseline (speedup 1.0000x reference)
import dataclasses
import functools

import jax
import jax.numpy as jnp
from jax import lax
from jax.experimental import pallas as pl
from jax.experimental.pallas import tpu as pltpu
from jax.experimental.pallas import tpu_sc as plsc

E = 8
D_IN = 768
D_OUT = 768
N_TOKENS = 4096
NK = 8192
TOP_K = NK // N_TOKENS
HALF = NK // 2

NC = 2
NS = 16
NW = NC * NS
CHUNK = 64

BLK = 512


def _sc_compiler_params():
    cp = pltpu.CompilerParams()
    if "needs_layout_passes" in pltpu.CompilerParams.__dataclass_fields__:
        cp = dataclasses.replace(cp, needs_layout_passes=False)
    return cp


@functools.cache
def _build_sc_gather(nrows, base):
    per_w = nrows // NW
    n_ch = per_w // CHUNK
    row0 = base // CHUNK
    mesh = plsc.VectorSubcoreMesh(core_axis_name="c", subcore_axis_name="s")

    @functools.partial(
        pl.kernel,
        mesh=mesh,
        compiler_params=_sc_compiler_params(),
        out_type=jax.ShapeDtypeStruct((nrows, D_IN), jnp.float32),
        scratch_types=[
            pltpu.VMEM((n_ch, CHUNK), jnp.int32),
            pltpu.VMEM((CHUNK, D_IN), jnp.float32),
            pltpu.VMEM((CHUNK, D_IN), jnp.float32),
            pltpu.SemaphoreType.DMA,
            pltpu.SemaphoreType.DMA,
            pltpu.SemaphoreType.DMA,
            pltpu.SemaphoreType.DMA,
        ],
    )
    def _gather(idx_hbm, src_hbm, out_hbm, idx_v, buf0, buf1, g0, g1, o0, o1):
        wid = lax.axis_index("s") * NC + lax.axis_index("c")
        loc = wid * per_w
        pltpu.sync_copy(idx_hbm.at[pl.ds(row0 + wid * n_ch, n_ch)], idx_v)
        for r in range(n_ch):
            for j in range(CHUNK // 16):
                sl = pl.ds(j * 16, 16)
                idx_v[r, sl] = idx_v[r, sl] // TOP_K
        bufs = (buf0, buf1)
        gsems = (g0, g1)
        osems = (o0, o1)
        gh = [None, None]
        oh = [None, None]
        for c in range(min(2, n_ch)):
            gh[c] = pltpu.async_copy(src_hbm.at[idx_v.at[c]], bufs[c],
                                     gsems[c])
        for c in range(n_ch):
            b = c % 2
            gh[b].wait()
            oh[b] = pltpu.async_copy(
                bufs[b], out_hbm.at[pl.ds(loc + c * CHUNK, CHUNK)], osems[b])
            nxt = c + 2
            if nxt < n_ch:
                oh[b].wait()
                gh[b] = pltpu.async_copy(src_hbm.at[idx_v.at[nxt]], bufs[b],
                                         gsems[b])
        for b in range(2):
            if oh[b] is not None:
                oh[b].wait()

    return _gather


@functools.cache
def _build_sc_scatter2():
    per_w = NK // NW
    n_ch = per_w // CHUNK
    mesh = plsc.VectorSubcoreMesh(core_axis_name="c", subcore_axis_name="s")

    @functools.partial(
        pl.kernel,
        mesh=mesh,
        compiler_params=_sc_compiler_params(),
        out_type=jax.ShapeDtypeStruct((NK, D_OUT), jnp.float32),
        scratch_types=[
            pltpu.VMEM((n_ch, CHUNK), jnp.int32),
            pltpu.VMEM((CHUNK, D_OUT), jnp.float32),
            pltpu.VMEM((CHUNK, D_OUT), jnp.float32),
            pltpu.SemaphoreType.DMA,
            pltpu.SemaphoreType.DMA,
            pltpu.SemaphoreType.DMA,
            pltpu.SemaphoreType.DMA,
        ],
    )
    def _scatter(idx_hbm, y0_hbm, y1_hbm, out_hbm, idx_v, buf0, buf1,
                 r0, r1, s0, s1):
        wid = lax.axis_index("s") * NC + lax.axis_index("c")
        pltpu.sync_copy(idx_hbm.at[pl.ds(wid * n_ch, n_ch)], idx_v)
        bufs = (buf0, buf1)
        rsems = (r0, r1)
        ssems = (s0, s1)

        def run(y_hbm, loc):
            rh = [None, None]
            sh = [None, None]
            for c in range(min(2, n_ch)):
                rh[c] = pltpu.async_copy(
                    y_hbm.at[pl.ds(loc + c * CHUNK, CHUNK)], bufs[c],
                    rsems[c])
            for c in range(n_ch):
                b = c % 2
                rh[b].wait()
                sh[b] = pltpu.async_copy(bufs[b], out_hbm.at[idx_v.at[c]],
                                         ssems[b])
                nxt = c + 2
                if nxt < n_ch:
                    sh[b].wait()
                    rh[b] = pltpu.async_copy(
                        y_hbm.at[pl.ds(loc + nxt * CHUNK, CHUNK)], bufs[b],
                        rsems[b])
            for b in range(2):
                if sh[b] is not None:
                    sh[b].wait()

        @pl.when(wid < NW // 2)
        def _():
            run(y0_hbm, wid * per_w)

        @pl.when(wid >= NW // 2)
        def _():
            run(y1_hbm, wid * per_w - HALF)

    return _scatter


def _mm_body(sei_ref, x_ref, w_ref, o_ref):
    e_lo = sei_ref[0, 0, 0]
    e_hi = sei_ref[0, 0, BLK - 1]
    o_ref[...] = jnp.zeros_like(o_ref)
    sei2 = sei_ref[0]
    row = lax.broadcasted_iota(jnp.int32, (BLK, 1), 0)
    for e in range(E):
        @pl.when((e >= e_lo) & (e <= e_hi))
        def _():
            lo = jnp.sum((sei2 < e).astype(jnp.int32))
            hi = jnp.sum((sei2 <= e).astype(jnp.int32))
            mask = (row >= lo) & (row < hi)
            xm = jnp.where(mask, x_ref[...], 0.0).astype(jnp.bfloat16)
            o_ref[...] += lax.dot_general(
                xm, w_ref[e],
                (((1,), (0,)), ((), ())),
                preferred_element_type=jnp.float32,
            )


def _grouped_mm(sei3, x_sorted, wt):
    n_tiles = x_sorted.shape[0] // BLK
    return pl.pallas_call(
        _mm_body,
        grid=(n_tiles,),
        in_specs=[
            pl.BlockSpec((1, 1, BLK), lambda i: (i, 0, 0)),
            pl.BlockSpec((BLK, D_IN), lambda i: (i, 0)),
            pl.BlockSpec((E, D_IN, D_OUT), lambda i: (0, 0, 0)),
        ],
        out_specs=pl.BlockSpec((BLK, D_OUT), lambda i: (i, 0)),
        out_shape=jax.ShapeDtypeStruct((x_sorted.shape[0], D_OUT), jnp.float32),
        compiler_params=pltpu.CompilerParams(
            dimension_semantics=("arbitrary",)),
    )(sei3, x_sorted, wt)


def kernel(inputs, weight, k, sorted_expert_idxs, sorted_scattered_idxs,
           padded_block_idxs):
    wt = jnp.transpose(weight, (0, 2, 1)).astype(jnp.bfloat16)
    ssi2 = sorted_scattered_idxs.reshape(NK // CHUNK, CHUNK)
    x0 = _build_sc_gather(HALF, 0)(ssi2, inputs)
    x1 = _build_sc_gather(HALF, HALF)(ssi2, inputs)
    sei3 = sorted_expert_idxs.reshape(NK // BLK, 1, BLK)
    y0 = _grouped_mm(sei3[: HALF // BLK], x0, wt)
    y1 = _grouped_mm(sei3[HALF // BLK:], x1, wt)
    return _build_sc_scatter2()(ssi2, y0, y1)


# speedup vs baseline: 1.0036x; 1.0036x over previous
import dataclasses
import functools

import jax
import jax.numpy as jnp
from jax import lax
from jax.experimental import pallas as pl
from jax.experimental.pallas import tpu as pltpu
from jax.experimental.pallas import tpu_sc as plsc

E = 8
D_IN = 768
D_OUT = 768
N_TOKENS = 4096
NK = 8192
TOP_K = NK // N_TOKENS
HALF = NK // 2

NC = 2
NS = 16
NW = NC * NS
CHUNK = 64

BLK = 512


def _sc_compiler_params():
    cp = pltpu.CompilerParams()
    if "needs_layout_passes" in pltpu.CompilerParams.__dataclass_fields__:
        cp = dataclasses.replace(cp, needs_layout_passes=False)
    return cp


@functools.cache
def _build_sc_gather(nrows, base):
    per_w = nrows // NW
    n_ch = per_w // CHUNK
    row0 = base // CHUNK
    mesh = plsc.VectorSubcoreMesh(core_axis_name="c", subcore_axis_name="s")

    @functools.partial(
        pl.kernel,
        mesh=mesh,
        compiler_params=_sc_compiler_params(),
        cost_estimate=pl.CostEstimate(flops=0, transcendentals=0,
                                      bytes_accessed=26_000_000),
        out_type=jax.ShapeDtypeStruct((nrows, D_IN), jnp.float32),
        scratch_types=[
            pltpu.VMEM((n_ch, CHUNK), jnp.int32),
            pltpu.VMEM((CHUNK, D_IN), jnp.float32),
            pltpu.VMEM((CHUNK, D_IN), jnp.float32),
            pltpu.SemaphoreType.DMA,
            pltpu.SemaphoreType.DMA,
            pltpu.SemaphoreType.DMA,
            pltpu.SemaphoreType.DMA,
        ],
    )
    def _gather(idx_hbm, src_hbm, out_hbm, idx_v, buf0, buf1, g0, g1, o0, o1):
        wid = lax.axis_index("s") * NC + lax.axis_index("c")
        loc = wid * per_w
        pltpu.sync_copy(idx_hbm.at[pl.ds(row0 + wid * n_ch, n_ch)], idx_v)
        for r in range(n_ch):
            for j in range(CHUNK // 16):
                sl = pl.ds(j * 16, 16)
                idx_v[r, sl] = idx_v[r, sl] // TOP_K
        bufs = (buf0, buf1)
        gsems = (g0, g1)
        osems = (o0, o1)
        gh = [None, None]
        oh = [None, None]
        for c in range(min(2, n_ch)):
            gh[c] = pltpu.async_copy(src_hbm.at[idx_v.at[c]], bufs[c],
                                     gsems[c])
        for c in range(n_ch):
            b = c % 2
            gh[b].wait()
            oh[b] = pltpu.async_copy(
                bufs[b], out_hbm.at[pl.ds(loc + c * CHUNK, CHUNK)], osems[b])
            nxt = c + 2
            if nxt < n_ch:
                oh[b].wait()
                gh[b] = pltpu.async_copy(src_hbm.at[idx_v.at[nxt]], bufs[b],
                                         gsems[b])
        for b in range(2):
            if oh[b] is not None:
                oh[b].wait()

    return _gather


@functools.cache
def _build_sc_scatter2():
    per_w = NK // NW
    n_ch = per_w // CHUNK
    mesh = plsc.VectorSubcoreMesh(core_axis_name="c", subcore_axis_name="s")

    @functools.partial(
        pl.kernel,
        mesh=mesh,
        compiler_params=_sc_compiler_params(),
        cost_estimate=pl.CostEstimate(flops=0, transcendentals=0,
                                      bytes_accessed=52_000_000),
        out_type=jax.ShapeDtypeStruct((NK, D_OUT), jnp.float32),
        scratch_types=[
            pltpu.VMEM((n_ch, CHUNK), jnp.int32),
            pltpu.VMEM((CHUNK, D_OUT), jnp.float32),
            pltpu.VMEM((CHUNK, D_OUT), jnp.float32),
            pltpu.SemaphoreType.DMA,
            pltpu.SemaphoreType.DMA,
            pltpu.SemaphoreType.DMA,
            pltpu.SemaphoreType.DMA,
        ],
    )
    def _scatter(idx_hbm, y0_hbm, y1_hbm, out_hbm, idx_v, buf0, buf1,
                 r0, r1, s0, s1):
        wid = lax.axis_index("s") * NC + lax.axis_index("c")
        pltpu.sync_copy(idx_hbm.at[pl.ds(wid * n_ch, n_ch)], idx_v)
        bufs = (buf0, buf1)
        rsems = (r0, r1)
        ssems = (s0, s1)

        def run(y_hbm, loc):
            rh = [None, None]
            sh = [None, None]
            for c in range(min(2, n_ch)):
                rh[c] = pltpu.async_copy(
                    y_hbm.at[pl.ds(loc + c * CHUNK, CHUNK)], bufs[c],
                    rsems[c])
            for c in range(n_ch):
                b = c % 2
                rh[b].wait()
                sh[b] = pltpu.async_copy(bufs[b], out_hbm.at[idx_v.at[c]],
                                         ssems[b])
                nxt = c + 2
                if nxt < n_ch:
                    sh[b].wait()
                    rh[b] = pltpu.async_copy(
                        y_hbm.at[pl.ds(loc + nxt * CHUNK, CHUNK)], bufs[b],
                        rsems[b])
            for b in range(2):
                if sh[b] is not None:
                    sh[b].wait()

        @pl.when(wid < NW // 2)
        def _():
            run(y0_hbm, wid * per_w)

        @pl.when(wid >= NW // 2)
        def _():
            run(y1_hbm, wid * per_w - HALF)

    return _scatter


def _mm_body(sei_ref, x_ref, w_ref, o_ref):
    e_lo = sei_ref[0, 0, 0]
    e_hi = sei_ref[0, 0, BLK - 1]
    o_ref[...] = jnp.zeros_like(o_ref)
    sei2 = sei_ref[0]
    row = lax.broadcasted_iota(jnp.int32, (BLK, 1), 0)
    for e in range(E):
        @pl.when((e >= e_lo) & (e <= e_hi))
        def _():
            lo = jnp.sum((sei2 < e).astype(jnp.int32))
            hi = jnp.sum((sei2 <= e).astype(jnp.int32))
            mask = (row >= lo) & (row < hi)
            xm = jnp.where(mask, x_ref[...], 0.0).astype(jnp.bfloat16)
            o_ref[...] += lax.dot_general(
                xm, w_ref[e],
                (((1,), (0,)), ((), ())),
                preferred_element_type=jnp.float32,
            )


def _grouped_mm(sei3, x_sorted, wt):
    n_tiles = x_sorted.shape[0] // BLK
    return pl.pallas_call(
        _mm_body,
        grid=(n_tiles,),
        in_specs=[
            pl.BlockSpec((1, 1, BLK), lambda i: (i, 0, 0)),
            pl.BlockSpec((BLK, D_IN), lambda i: (i, 0)),
            pl.BlockSpec((E, D_IN, D_OUT), lambda i: (0, 0, 0)),
        ],
        out_specs=pl.BlockSpec((BLK, D_OUT), lambda i: (i, 0)),
        out_shape=jax.ShapeDtypeStruct((x_sorted.shape[0], D_OUT), jnp.float32),
        compiler_params=pltpu.CompilerParams(
            dimension_semantics=("arbitrary",)),
        cost_estimate=pl.CostEstimate(flops=5_000_000_000,
                                      transcendentals=0,
                                      bytes_accessed=31_000_000),
    )(sei3, x_sorted, wt)


def kernel(inputs, weight, k, sorted_expert_idxs, sorted_scattered_idxs,
           padded_block_idxs):
    wt = jnp.transpose(weight, (0, 2, 1)).astype(jnp.bfloat16)
    ssi2 = sorted_scattered_idxs.reshape(NK // CHUNK, CHUNK)
    x0 = _build_sc_gather(HALF, 0)(ssi2, inputs)
    x1 = _build_sc_gather(HALF, HALF)(ssi2, inputs)
    sei3 = sorted_expert_idxs.reshape(NK // BLK, 1, BLK)
    y0 = _grouped_mm(sei3[: HALF // BLK], x0, wt)
    y1 = _grouped_mm(sei3[HALF // BLK:], x1, wt)
    return _build_sc_scatter2()(ssi2, y0, y1)
